# baseline (speedup 1.0000x reference)
import functools
import numpy as np
import jax
import jax.numpy as jnp
from jax import lax
from jax.experimental import pallas as pl
from jax.experimental.pallas import tpu as pltpu

NB = 8


def _ln_last(t, w, b):
    mu = jnp.mean(t, axis=-1, keepdims=True)
    var = jnp.mean(jnp.square(t - mu), axis=-1, keepdims=True)
    return (t - mu) * lax.rsqrt(var + 1e-5) * w + b


def _l2n_rows(v):
    ss = jnp.sum(v * v, axis=-1, keepdims=True)
    return v * lax.rsqrt(jnp.maximum(ss, 1e-24))


def _softmax_rows(s):
    m = jnp.max(s, axis=-1, keepdims=True)
    e = jnp.exp(s - m)
    return e * pl.reciprocal(jnp.sum(e, axis=-1, keepdims=True), approx=True)


def _erf(x):
    a1, a2, a3, a4, a5 = 0.254829592, -0.284496736, 1.421413741, -1.453152027, 1.061405429
    p = 0.3275911
    s = jnp.sign(x)
    z = jnp.abs(x)
    t = pl.reciprocal(1.0 + p * z, approx=True)
    poly = t * (a1 + t * (a2 + t * (a3 + t * (a4 + t * a5))))
    return s * (1.0 - poly * jnp.exp(-z * z))


def _gelu(x):
    return 0.5 * x * (1.0 + _erf(x * 0.7071067811865476))


def _bdot(a, b):
    return jnp.dot(a.astype(jnp.bfloat16), b.astype(jnp.bfloat16),
                   preferred_element_type=jnp.float32)


def _bdot_t(a, b):
    return lax.dot_general(a.astype(jnp.bfloat16), b.astype(jnp.bfloat16),
                           dimension_numbers=(((1,), (1,)), ((), ())),
                           preferred_element_type=jnp.float32)


def _patch9_multi(t2, cb, lb, rb, patch, mL, mR, HW, W, C):
    for i in range(NB):
        t = t2[i * HW:(i + 1) * HW, :]
        zhalo = jnp.zeros((W, C), jnp.bfloat16)
        cb[0:W, :] = zhalo
        cb[W + HW:2 * W + HW, :] = zhalo
        lb[0:W, :] = zhalo
        lb[W + HW:2 * W + HW, :] = zhalo
        rb[0:W, :] = zhalo
        rb[W + HW:2 * W + HW, :] = zhalo
        cb[W:W + HW, :] = t.astype(jnp.bfloat16)
        lb[W:W + HW, :] = cb[W - 1:W - 1 + HW, :] * mL
        rb[W:W + HW, :] = cb[W + 1:W + 1 + HW, :] * mR
        r0 = i * HW
        for ki in range(3):
            o = W * ki
            patch[r0:r0 + HW, (3 * ki + 0) * C:(3 * ki + 1) * C] = lb[o:o + HW, :]
            patch[r0:r0 + HW, (3 * ki + 1) * C:(3 * ki + 2) * C] = cb[o:o + HW, :]
            patch[r0:r0 + HW, (3 * ki + 2) * C:(3 * ki + 3) * C] = rb[o:o + HW, :]


def _main_kernel(
        x_ref, y_ref, selT_ref,
        lnxw_ref, lnxb_ref, lnyw_ref, lnyb_ref,
        wqx_ref, wqy_ref,
        t1r_ref, t2r_ref, t3r_ref, hmask_ref,
        wproj_ref, nw_ref, nb_ref,
        wfi_ref, wfo_ref,
        wfz_ref, bfz_ref,
        z_ref, stats_ref,
        cb, lb, rb, patch,
        *, H, W):
    C = x_ref.shape[1]
    HW = x_ref.shape[2]
    M = NB * HW
    hid2 = wfi_ref.shape[1]
    hid = hid2 // 2

    col = lax.broadcasted_iota(jnp.int32, (HW, 1), 0) % W
    mL = (col != 0).astype(jnp.bfloat16)
    mR = (col != W - 1).astype(jnp.bfloat16)

    x_cl = jnp.transpose(x_ref[...], (0, 2, 1)).reshape(M, C)
    y_cl = jnp.transpose(y_ref[...], (0, 2, 1)).reshape(M, C)

    def branch(t_cl, lnw, lnb, wq_ref):
        tn = _ln_last(t_cl, lnw, lnb)
        _patch9_multi(tn, cb, lb, rb, patch, mL, mR, HW, W, C)
        return jnp.dot(patch[...], wq_ref[...],
                       preferred_element_type=jnp.float32)

    dwx = branch(x_cl, lnxw_ref[...], lnxb_ref[...], wqx_ref)
    dwy = branch(y_cl, lnyw_ref[...], lnyb_ref[...], wqy_ref)

    selT = selT_ref[...]
    hmask = hmask_ref[...]
    t1r = t1r_ref[...]
    t2r = t2r_ref[...]
    t3r = t3r_ref[...]

    attn_rows = []
    for i in range(NB):
        dx = jnp.transpose(dwx[i * HW:(i + 1) * HW, :])
        dy = jnp.transpose(dwy[i * HW:(i + 1) * HW, :])
        qx, kx, vx = dx[0:C], dx[C:2 * C], dx[2 * C:3 * C]
        qy, ky, vy = dy[0:C], dy[C:2 * C], dy[2 * C:3 * C]

        kxs = _bdot(kx, selT)
        vxs = _bdot(vx, selT)
        qys = _bdot(qy, selT)

        qxn = _l2n_rows(qx)
        kyn = _l2n_rows(ky)
        qyn = _l2n_rows(qys)
        kxn = _l2n_rows(kxs)

        s1 = _bdot_t(qxn, kyn) * t1r
        attnx = _softmax_rows(jnp.where(hmask > 0.5, s1, -1e30))
        s2 = _bdot_t(qyn, kxn) * t2r
        attny = _softmax_rows(jnp.where(hmask > 0.5, s2, -1e30))

        av = _bdot(attny, vxs)
        t2v = _bdot(attnx, av)
        s3 = _bdot_t(t2v, vxs) * hmask * t3r
        attn = _bdot(s3, vy)
        attn_rows.append(jnp.transpose(attn))

    attn_cl = jnp.concatenate(attn_rows, axis=0)

    out = x_cl + _bdot(attn_cl, wproj_ref[...])

    on = _ln_last(out, nw_ref[...], nb_ref[...])
    _patch9_multi(on, cb, lb, rb, patch, mL, mR, HW, W, C)
    dwf = jnp.dot(patch[...], wfi_ref[...],
                  preferred_element_type=jnp.float32)
    g = _gelu(dwf[:, 0:hid]) * dwf[:, hid:hid2]
    out = out + _bdot(g, wfo_ref[...])

    z0 = x_cl + x_cl * out
    _patch9_multi(z0, cb, lb, rb, patch, mL, mR, HW, W, C)
    z2 = jnp.dot(patch[...], wfz_ref[...],
                 preferred_element_type=jnp.float32)

    bfz = bfz_ref[...]
    for i in range(NB):
        zi = z2[i * HW:(i + 1) * HW, :] + bfz
        z_ref[i] = jnp.transpose(zi)
        s_sum = jnp.sum(zi, axis=0, keepdims=True)
        s_sq = jnp.sum(zi * zi, axis=0, keepdims=True)
        stats_ref[i] = jnp.concatenate(
            [s_sum, s_sq, jnp.zeros((6, C), jnp.float32)], axis=0)


def _bn_relu_kernel(z_ref, scale_ref, shift_ref, o_ref):
    o_ref[...] = jnp.maximum(z_ref[...] * scale_ref[...] + shift_ref[...], 0.0)


def kernel(x, y, ln_x_w, ln_x_b, ln_y_w, ln_y_b, w_qkv_x, w_qkv_y, w_dw_x,
           w_dw_y, t1, t2, t3, w_proj, norm_w, norm_b, w_ffn_in, w_ffn_dw,
           w_ffn_out, w_fuse1, b_fuse1, w_fuse2, b_fuse2, bn_w, bn_b):
    B, C, H, W = x.shape
    HW = H * W
    Ho, Wo = (H + 1) // 2, (W + 1) // 2
    HWs = Ho * Wo
    C3 = 3 * C
    hid = w_ffn_out.shape[0]
    hid2 = 2 * hid
    num_heads = t1.shape[0]
    hc = C // num_heads

    x2 = x.reshape(B, C, HW)
    y2 = y.reshape(B, C, HW)

    sel = np.zeros((HW, HWs), np.float32)
    pos = (2 * (np.arange(HWs) // Wo)) * W + 2 * (np.arange(HWs) % Wo)
    sel[pos, np.arange(HWs)] = 1.0
    selT = jnp.asarray(sel, jnp.bfloat16)

    hm = (np.arange(C)[:, None] // hc == np.arange(C)[None, :] // hc)
    hmask = jnp.asarray(hm.astype(np.float32))
    t1r = jnp.repeat(t1.reshape(num_heads), hc).reshape(C, 1)
    t2r = jnp.repeat(t2.reshape(num_heads), hc).reshape(C, 1)
    t3r = jnp.repeat(t3.reshape(num_heads), hc).reshape(C, 1)

    bf16 = jnp.bfloat16

    def dw_compose(w1, wdw):
        return jnp.concatenate(
            [w1 * wdw[k][None, :] for k in range(9)], axis=0).astype(bf16)

    wqx_c = dw_compose(w_qkv_x, w_dw_x)
    wqy_c = dw_compose(w_qkv_y, w_dw_y)
    wfi_c = dw_compose(w_ffn_in, w_ffn_dw)

    wfz_c = jnp.concatenate(
        [w_fuse1 @ w_fuse2[k] for k in range(9)], axis=0).astype(bf16)
    r_idx = np.arange(HW) // W
    c_idx = np.arange(HW) % W
    bias_rows = []
    for k in range(9):
        ki, kj = k // 3, k % 3
        valid = ((r_idx + ki - 1 >= 0) & (r_idx + ki - 1 < H)
                 & (c_idx + kj - 1 >= 0) & (c_idx + kj - 1 < W))
        bias_rows.append(valid.astype(np.float32))
    validity = jnp.asarray(np.stack(bias_rows, axis=1))
    tap_bias = jnp.stack([(b_fuse1 @ w_fuse2[k]).reshape(-1)
                          for k in range(9)], axis=0)
    bfz_field = validity @ tap_bias + b_fuse2

    wproj_b = w_proj.astype(bf16)
    wfo_b = w_ffn_out.astype(bf16)

    wspec = lambda *shape: pl.BlockSpec(shape, lambda b, s=shape: (0,) * len(s))
    nbspec = lambda *shape: pl.BlockSpec((NB,) + shape,
                                         lambda b, s=shape: (b,) + (0,) * len(s))

    kfn = functools.partial(_main_kernel, H=H, W=W)
    pad = HW + 2 * W

    z, stats = pl.pallas_call(
        kfn,
        out_shape=[jax.ShapeDtypeStruct((B, C, HW), jnp.float32),
                   jax.ShapeDtypeStruct((B, 8, C), jnp.float32)],
        grid=(B // NB,),
        in_specs=[
            nbspec(C, HW), nbspec(C, HW), wspec(HW, HWs),
            wspec(1, C), wspec(1, C), wspec(1, C), wspec(1, C),
            wspec(9 * C, C3), wspec(9 * C, C3),
            wspec(C, 1), wspec(C, 1), wspec(C, 1), wspec(C, C),
            wspec(C, C), wspec(1, C), wspec(1, C),
            wspec(9 * C, hid2), wspec(hid, C),
            wspec(9 * C, C), wspec(HW, C),
        ],
        out_specs=[nbspec(C, HW), nbspec(8, C)],
        scratch_shapes=[
            pltpu.VMEM((pad, C), jnp.bfloat16),
            pltpu.VMEM((pad, C), jnp.bfloat16),
            pltpu.VMEM((pad, C), jnp.bfloat16),
            pltpu.VMEM((NB * HW, 9 * C), jnp.bfloat16),
        ],
        compiler_params=pltpu.CompilerParams(
            dimension_semantics=("parallel",),
            vmem_limit_bytes=48 * 1024 * 1024),
    )(x2, y2, selT,
      ln_x_w, ln_x_b, ln_y_w, ln_y_b,
      wqx_c, wqy_c,
      t1r, t2r, t3r, hmask,
      wproj_b, norm_w, norm_b,
      wfi_c, wfo_b,
      wfz_c, bfz_field)

    n = B * HW
    mean = jnp.sum(stats[:, 0, :], axis=0) / n
    var = jnp.maximum(jnp.sum(stats[:, 1, :], axis=0) / n - mean * mean, 0.0)
    inv = lax.rsqrt(var + 1e-5)
    bw = bn_w.reshape(-1)
    bb = bn_b.reshape(-1)
    scale = (bw * inv).reshape(C, 1)
    shift = (bb - mean * bw * inv).reshape(C, 1)

    out = pl.pallas_call(
        _bn_relu_kernel,
        out_shape=jax.ShapeDtypeStruct((B, C, HW), jnp.float32),
        grid=(B,),
        in_specs=[pl.BlockSpec((None, C, HW), lambda b: (b, 0, 0)),
                  pl.BlockSpec((C, 1), lambda b: (0, 0)),
                  pl.BlockSpec((C, 1), lambda b: (0, 0))],
        out_specs=pl.BlockSpec((None, C, HW), lambda b: (b, 0, 0)),
        compiler_params=pltpu.CompilerParams(dimension_semantics=("parallel",)),
    )(z, scale, shift)

    return out.reshape(B, C, H, W)


# speedup vs baseline: 1.0570x; 1.0570x over previous
import functools
import numpy as np
import jax
import jax.numpy as jnp
from jax import lax
from jax.experimental import pallas as pl
from jax.experimental.pallas import tpu as pltpu

NB = 4


def _ln_last(t, w, b):
    mu = jnp.mean(t, axis=-1, keepdims=True)
    var = jnp.mean(jnp.square(t - mu), axis=-1, keepdims=True)
    return (t - mu) * lax.rsqrt(var + 1e-5) * w + b


def _l2n_rows(v):
    ss = jnp.sum(v * v, axis=-1, keepdims=True)
    return v * lax.rsqrt(jnp.maximum(ss, 1e-24))


def _softmax_rows(s):
    m = jnp.max(s, axis=-1, keepdims=True)
    e = jnp.exp(s - m)
    return e * pl.reciprocal(jnp.sum(e, axis=-1, keepdims=True), approx=True)


def _gelu(x):
    x2 = x * x
    u2 = x * (1.5957691216057308 + 0.07135481627613564 * x2)
    r = pl.reciprocal(jnp.exp(u2) + 1.0, approx=True)
    return x * (1.0 - r)


def _bdot(a, b):
    return jnp.dot(a.astype(jnp.bfloat16), b.astype(jnp.bfloat16),
                   preferred_element_type=jnp.float32)


def _bdot_t(a, b):
    return lax.dot_general(a.astype(jnp.bfloat16), b.astype(jnp.bfloat16),
                           dimension_numbers=(((1,), (1,)), ((), ())),
                           preferred_element_type=jnp.float32)


def _patch9_multi(t2, cb, lb, rb, patch, mL, mR, HW, W, C):
    for i in range(NB):
        t = t2[i * HW:(i + 1) * HW, :]
        zhalo = jnp.zeros((W, C), jnp.bfloat16)
        cb[0:W, :] = zhalo
        cb[W + HW:2 * W + HW, :] = zhalo
        lb[0:W, :] = zhalo
        lb[W + HW:2 * W + HW, :] = zhalo
        rb[0:W, :] = zhalo
        rb[W + HW:2 * W + HW, :] = zhalo
        cb[W:W + HW, :] = t.astype(jnp.bfloat16)
        lb[W:W + HW, :] = cb[W - 1:W - 1 + HW, :] * mL
        rb[W:W + HW, :] = cb[W + 1:W + 1 + HW, :] * mR
        r0 = i * HW
        for ki in range(3):
            o = W * ki
            patch[r0:r0 + HW, (3 * ki + 0) * C:(3 * ki + 1) * C] = lb[o:o + HW, :]
            patch[r0:r0 + HW, (3 * ki + 1) * C:(3 * ki + 2) * C] = cb[o:o + HW, :]
            patch[r0:r0 + HW, (3 * ki + 2) * C:(3 * ki + 3) * C] = rb[o:o + HW, :]


def _main_kernel(
        x_ref, y_ref, selT_ref,
        lnxw_ref, lnxb_ref, lnyw_ref, lnyb_ref,
        wqx_ref, wqy_ref,
        t1r_ref, t2r_ref, t3r_ref, hmask_ref,
        wproj_ref, nw_ref, nb_ref,
        wfi_ref, wfo_ref,
        wfz_ref, bfz_ref,
        z_ref, stats_ref,
        cba, lba, rba, patcha, cbb, lbb, rbb, patchb,
        *, H, W):
    C = x_ref.shape[1]
    HW = x_ref.shape[2]
    M = NB * HW
    hid2 = wfi_ref.shape[1]
    hid = hid2 // 2

    col = lax.broadcasted_iota(jnp.int32, (HW, 1), 0) % W
    mL = (col != 0).astype(jnp.bfloat16)
    mR = (col != W - 1).astype(jnp.bfloat16)

    x_cl = jnp.transpose(x_ref[...], (0, 2, 1)).reshape(M, C)
    y_cl = jnp.transpose(y_ref[...], (0, 2, 1)).reshape(M, C)

    def branch(t_cl, lnw, lnb, wq_ref, cb, lb, rb, patch):
        tn = _ln_last(t_cl, lnw, lnb)
        _patch9_multi(tn, cb, lb, rb, patch, mL, mR, HW, W, C)
        return jnp.dot(patch[...], wq_ref[...],
                       preferred_element_type=jnp.float32)

    dwx = branch(x_cl, lnxw_ref[...], lnxb_ref[...], wqx_ref,
                 cba, lba, rba, patcha)
    dwy = branch(y_cl, lnyw_ref[...], lnyb_ref[...], wqy_ref,
                 cbb, lbb, rbb, patchb)

    selT = selT_ref[...]
    hmask = hmask_ref[...]
    t1r = t1r_ref[...]
    t2r = t2r_ref[...]
    t3r = t3r_ref[...]

    attn_rows = []
    for i in range(NB):
        dx = jnp.transpose(dwx[i * HW:(i + 1) * HW, :])
        dy = jnp.transpose(dwy[i * HW:(i + 1) * HW, :])
        qx, kx, vx = dx[0:C], dx[C:2 * C], dx[2 * C:3 * C]
        qy, ky, vy = dy[0:C], dy[C:2 * C], dy[2 * C:3 * C]

        kxs = _bdot(kx, selT)
        vxs = _bdot(vx, selT)
        qys = _bdot(qy, selT)

        qxn = _l2n_rows(qx)
        kyn = _l2n_rows(ky)
        qyn = _l2n_rows(qys)
        kxn = _l2n_rows(kxs)

        s1 = _bdot_t(qxn, kyn) * t1r
        attnx = _softmax_rows(jnp.where(hmask > 0.5, s1, -1e30))
        s2 = _bdot_t(qyn, kxn) * t2r
        attny = _softmax_rows(jnp.where(hmask > 0.5, s2, -1e30))

        av = _bdot(attny, vxs)
        t2v = _bdot(attnx, av)
        s3 = _bdot_t(t2v, vxs) * hmask * t3r
        attn = _bdot(s3, vy)
        attn_rows.append(jnp.transpose(attn))

    attn_cl = jnp.concatenate(attn_rows, axis=0)

    out = x_cl + _bdot(attn_cl, wproj_ref[...])

    on = _ln_last(out, nw_ref[...], nb_ref[...])
    _patch9_multi(on, cba, lba, rba, patcha, mL, mR, HW, W, C)
    dwf = jnp.dot(patcha[...], wfi_ref[...],
                  preferred_element_type=jnp.float32)
    g = _gelu(dwf[:, 0:hid]) * dwf[:, hid:hid2]
    out = out + _bdot(g, wfo_ref[...])

    z0 = x_cl + x_cl * out
    _patch9_multi(z0, cbb, lbb, rbb, patchb, mL, mR, HW, W, C)
    z2 = jnp.dot(patchb[...], wfz_ref[...],
                 preferred_element_type=jnp.float32)

    bfz = bfz_ref[...]
    for i in range(NB):
        zi = z2[i * HW:(i + 1) * HW, :] + bfz
        z_ref[i] = jnp.transpose(zi)
        s_sum = jnp.sum(zi, axis=0, keepdims=True)
        s_sq = jnp.sum(zi * zi, axis=0, keepdims=True)
        stats_ref[i] = jnp.concatenate(
            [s_sum, s_sq, jnp.zeros((6, C), jnp.float32)], axis=0)


def _bn_relu_kernel(z_ref, scale_ref, shift_ref, o_ref):
    o_ref[...] = jnp.maximum(z_ref[...] * scale_ref[...] + shift_ref[...], 0.0)


def kernel(x, y, ln_x_w, ln_x_b, ln_y_w, ln_y_b, w_qkv_x, w_qkv_y, w_dw_x,
           w_dw_y, t1, t2, t3, w_proj, norm_w, norm_b, w_ffn_in, w_ffn_dw,
           w_ffn_out, w_fuse1, b_fuse1, w_fuse2, b_fuse2, bn_w, bn_b):
    B, C, H, W = x.shape
    HW = H * W
    Ho, Wo = (H + 1) // 2, (W + 1) // 2
    HWs = Ho * Wo
    C3 = 3 * C
    hid = w_ffn_out.shape[0]
    hid2 = 2 * hid
    num_heads = t1.shape[0]
    hc = C // num_heads

    x2 = x.reshape(B, C, HW)
    y2 = y.reshape(B, C, HW)

    sel = np.zeros((HW, HWs), np.float32)
    pos = (2 * (np.arange(HWs) // Wo)) * W + 2 * (np.arange(HWs) % Wo)
    sel[pos, np.arange(HWs)] = 1.0
    selT = jnp.asarray(sel, jnp.bfloat16)

    hm = (np.arange(C)[:, None] // hc == np.arange(C)[None, :] // hc)
    hmask = jnp.asarray(hm.astype(np.float32))
    t1r = jnp.repeat(t1.reshape(num_heads), hc).reshape(C, 1)
    t2r = jnp.repeat(t2.reshape(num_heads), hc).reshape(C, 1)
    t3r = jnp.repeat(t3.reshape(num_heads), hc).reshape(C, 1)

    bf16 = jnp.bfloat16

    def dw_compose(w1, wdw):
        return jnp.concatenate(
            [w1 * wdw[k][None, :] for k in range(9)], axis=0).astype(bf16)

    wqx_c = dw_compose(w_qkv_x, w_dw_x)
    wqy_c = dw_compose(w_qkv_y, w_dw_y)
    wfi_c = dw_compose(w_ffn_in, w_ffn_dw)

    wfz_c = jnp.concatenate(
        [w_fuse1 @ w_fuse2[k] for k in range(9)], axis=0).astype(bf16)
    r_idx = np.arange(HW) // W
    c_idx = np.arange(HW) % W
    bias_rows = []
    for k in range(9):
        ki, kj = k // 3, k % 3
        valid = ((r_idx + ki - 1 >= 0) & (r_idx + ki - 1 < H)
                 & (c_idx + kj - 1 >= 0) & (c_idx + kj - 1 < W))
        bias_rows.append(valid.astype(np.float32))
    validity = jnp.asarray(np.stack(bias_rows, axis=1))
    tap_bias = jnp.stack([(b_fuse1 @ w_fuse2[k]).reshape(-1)
                          for k in range(9)], axis=0)
    bfz_field = validity @ tap_bias + b_fuse2

    wproj_b = w_proj.astype(bf16)
    wfo_b = w_ffn_out.astype(bf16)

    wspec = lambda *shape: pl.BlockSpec(shape, lambda b, s=shape: (0,) * len(s))
    nbspec = lambda *shape: pl.BlockSpec((NB,) + shape,
                                         lambda b, s=shape: (b,) + (0,) * len(s))

    kfn = functools.partial(_main_kernel, H=H, W=W)
    pad = HW + 2 * W

    z, stats = pl.pallas_call(
        kfn,
        out_shape=[jax.ShapeDtypeStruct((B, C, HW), jnp.float32),
                   jax.ShapeDtypeStruct((B, 8, C), jnp.float32)],
        grid=(B // NB,),
        in_specs=[
            nbspec(C, HW), nbspec(C, HW), wspec(HW, HWs),
            wspec(1, C), wspec(1, C), wspec(1, C), wspec(1, C),
            wspec(9 * C, C3), wspec(9 * C, C3),
            wspec(C, 1), wspec(C, 1), wspec(C, 1), wspec(C, C),
            wspec(C, C), wspec(1, C), wspec(1, C),
            wspec(9 * C, hid2), wspec(hid, C),
            wspec(9 * C, C), wspec(HW, C),
        ],
        out_specs=[nbspec(C, HW), nbspec(8, C)],
        scratch_shapes=[
            pltpu.VMEM((pad, C), jnp.bfloat16),
            pltpu.VMEM((pad, C), jnp.bfloat16),
            pltpu.VMEM((pad, C), jnp.bfloat16),
            pltpu.VMEM((NB * HW, 9 * C), jnp.bfloat16),
            pltpu.VMEM((pad, C), jnp.bfloat16),
            pltpu.VMEM((pad, C), jnp.bfloat16),
            pltpu.VMEM((pad, C), jnp.bfloat16),
            pltpu.VMEM((NB * HW, 9 * C), jnp.bfloat16),
        ],
        compiler_params=pltpu.CompilerParams(
            dimension_semantics=("parallel",),
            vmem_limit_bytes=48 * 1024 * 1024),
    )(x2, y2, selT,
      ln_x_w, ln_x_b, ln_y_w, ln_y_b,
      wqx_c, wqy_c,
      t1r, t2r, t3r, hmask,
      wproj_b, norm_w, norm_b,
      wfi_c, wfo_b,
      wfz_c, bfz_field)

    n = B * HW
    mean = jnp.sum(stats[:, 0, :], axis=0) / n
    var = jnp.maximum(jnp.sum(stats[:, 1, :], axis=0) / n - mean * mean, 0.0)
    inv = lax.rsqrt(var + 1e-5)
    bw = bn_w.reshape(-1)
    bb = bn_b.reshape(-1)
    scale = (bw * inv).reshape(C, 1)
    shift = (bb - mean * bw * inv).reshape(C, 1)

    out = pl.pallas_call(
        _bn_relu_kernel,
        out_shape=jax.ShapeDtypeStruct((B, C, HW), jnp.float32),
        grid=(B,),
        in_specs=[pl.BlockSpec((None, C, HW), lambda b: (b, 0, 0)),
                  pl.BlockSpec((C, 1), lambda b: (0, 0)),
                  pl.BlockSpec((C, 1), lambda b: (0, 0))],
        out_specs=pl.BlockSpec((None, C, HW), lambda b: (b, 0, 0)),
        compiler_params=pltpu.CompilerParams(dimension_semantics=("parallel",)),
    )(z, scale, shift)

    return out.reshape(B, C, H, W)


# speedup vs baseline: 1.0572x; 1.0003x over previous
import functools
import numpy as np
import jax
import jax.numpy as jnp
from jax import lax
from jax.experimental import pallas as pl
from jax.experimental.pallas import tpu as pltpu

NB = 4


def _ln_last(t, w, b):
    mu = jnp.mean(t, axis=-1, keepdims=True)
    var = jnp.mean(jnp.square(t - mu), axis=-1, keepdims=True)
    return (t - mu) * lax.rsqrt(var + 1e-5) * w + b


def _l2n_cols(v):
    ss = jnp.sum(v * v, axis=0, keepdims=True)
    return v * lax.rsqrt(jnp.maximum(ss, 1e-24))


def _softmax_rows(s):
    m = jnp.max(s, axis=-1, keepdims=True)
    e = jnp.exp(s - m)
    return e * pl.reciprocal(jnp.sum(e, axis=-1, keepdims=True), approx=True)


def _gelu(x):
    x2 = x * x
    u2 = x * (1.5957691216057308 + 0.07135481627613564 * x2)
    r = pl.reciprocal(jnp.exp(u2) + 1.0, approx=True)
    return x * (1.0 - r)


def _bdot(a, b):
    return jnp.dot(a.astype(jnp.bfloat16), b.astype(jnp.bfloat16),
                   preferred_element_type=jnp.float32)


def _bdot_t(a, b):
    return lax.dot_general(a.astype(jnp.bfloat16), b.astype(jnp.bfloat16),
                           dimension_numbers=(((1,), (1,)), ((), ())),
                           preferred_element_type=jnp.float32)


def _bdot_0(a, b):
    return lax.dot_general(a.astype(jnp.bfloat16), b.astype(jnp.bfloat16),
                           dimension_numbers=(((0,), (0,)), ((), ())),
                           preferred_element_type=jnp.float32)


def _patch9_multi(t2, cb, lb, rb, patch, mL, mR, HW, W, C):
    for i in range(NB):
        t = t2[i * HW:(i + 1) * HW, :]
        zhalo = jnp.zeros((W, C), jnp.bfloat16)
        cb[0:W, :] = zhalo
        cb[W + HW:2 * W + HW, :] = zhalo
        lb[0:W, :] = zhalo
        lb[W + HW:2 * W + HW, :] = zhalo
        rb[0:W, :] = zhalo
        rb[W + HW:2 * W + HW, :] = zhalo
        cb[W:W + HW, :] = t.astype(jnp.bfloat16)
        lb[W:W + HW, :] = cb[W - 1:W - 1 + HW, :] * mL
        rb[W:W + HW, :] = cb[W + 1:W + 1 + HW, :] * mR
        r0 = i * HW
        for ki in range(3):
            o = W * ki
            patch[r0:r0 + HW, (3 * ki + 0) * C:(3 * ki + 1) * C] = lb[o:o + HW, :]
            patch[r0:r0 + HW, (3 * ki + 1) * C:(3 * ki + 2) * C] = cb[o:o + HW, :]
            patch[r0:r0 + HW, (3 * ki + 2) * C:(3 * ki + 3) * C] = rb[o:o + HW, :]


def _main_kernel(
        x_ref, y_ref, selT_ref,
        lnxw_ref, lnxb_ref, lnyw_ref, lnyb_ref,
        wqx_ref, wqy_ref,
        t1r_ref, t2r_ref, t3r_ref, hmask_ref,
        wproj_ref, nw_ref, nb_ref,
        wfi_ref, wfo_ref,
        wfz_ref, bfz_ref,
        z_ref, stats_ref,
        cba, lba, rba, patcha, cbb, lbb, rbb, patchb,
        *, H, W):
    C = x_ref.shape[1]
    HW = x_ref.shape[2]
    M = NB * HW
    hid2 = wfi_ref.shape[1]
    hid = hid2 // 2

    col = lax.broadcasted_iota(jnp.int32, (HW, 1), 0) % W
    mL = (col != 0).astype(jnp.bfloat16)
    mR = (col != W - 1).astype(jnp.bfloat16)

    x_cl = jnp.transpose(x_ref[...], (0, 2, 1)).reshape(M, C)
    y_cl = jnp.transpose(y_ref[...], (0, 2, 1)).reshape(M, C)

    def branch(t_cl, lnw, lnb, wq_ref, cb, lb, rb, patch):
        tn = _ln_last(t_cl, lnw, lnb)
        _patch9_multi(tn, cb, lb, rb, patch, mL, mR, HW, W, C)
        return jnp.dot(patch[...], wq_ref[...],
                       preferred_element_type=jnp.float32)

    dwx = branch(x_cl, lnxw_ref[...], lnxb_ref[...], wqx_ref,
                 cba, lba, rba, patcha)
    dwy = branch(y_cl, lnyw_ref[...], lnyb_ref[...], wqy_ref,
                 cbb, lbb, rbb, patchb)

    selT = selT_ref[...]
    hmask = hmask_ref[...]
    t1r = t1r_ref[...]
    t2r = t2r_ref[...]
    t3r = t3r_ref[...]

    attn_rows = []
    for i in range(NB):
        dx = dwx[i * HW:(i + 1) * HW, :]
        dy = dwy[i * HW:(i + 1) * HW, :]
        qx, kx, vx = dx[:, 0:C], dx[:, C:2 * C], dx[:, 2 * C:3 * C]
        qy, ky, vy = dy[:, 0:C], dy[:, C:2 * C], dy[:, 2 * C:3 * C]

        kxs = _bdot_0(selT, kx)
        vxs = _bdot_0(selT, vx)
        qys = _bdot_0(selT, qy)

        qxn = _l2n_cols(qx)
        kyn = _l2n_cols(ky)
        qyn = _l2n_cols(qys)
        kxn = _l2n_cols(kxs)

        s1 = _bdot_0(qxn, kyn) * t1r
        attnx = _softmax_rows(jnp.where(hmask > 0.5, s1, -1e30))
        s2 = _bdot_0(qyn, kxn) * t2r
        attny = _softmax_rows(jnp.where(hmask > 0.5, s2, -1e30))

        av = _bdot_t(vxs, attny)
        t2v = _bdot_t(av, attnx)
        s3 = _bdot_0(t2v, vxs) * hmask * t3r
        attn_rows.append(_bdot_t(vy, s3))

    attn_cl = jnp.concatenate(attn_rows, axis=0)

    out = x_cl + _bdot(attn_cl, wproj_ref[...])

    on = _ln_last(out, nw_ref[...], nb_ref[...])
    _patch9_multi(on, cba, lba, rba, patcha, mL, mR, HW, W, C)
    dwf = jnp.dot(patcha[...], wfi_ref[...],
                  preferred_element_type=jnp.float32)
    g = _gelu(dwf[:, 0:hid]) * dwf[:, hid:hid2]
    out = out + _bdot(g, wfo_ref[...])

    z0 = x_cl + x_cl * out
    _patch9_multi(z0, cbb, lbb, rbb, patchb, mL, mR, HW, W, C)
    z2 = jnp.dot(patchb[...], wfz_ref[...],
                 preferred_element_type=jnp.float32)

    bfz = bfz_ref[...]
    for i in range(NB):
        zi = z2[i * HW:(i + 1) * HW, :] + bfz
        z_ref[i] = jnp.transpose(zi)
        s_sum = jnp.sum(zi, axis=0, keepdims=True)
        s_sq = jnp.sum(zi * zi, axis=0, keepdims=True)
        stats_ref[i] = jnp.concatenate(
            [s_sum, s_sq, jnp.zeros((6, C), jnp.float32)], axis=0)


def _bn_relu_kernel(z_ref, scale_ref, shift_ref, o_ref):
    o_ref[...] = jnp.maximum(z_ref[...] * scale_ref[...] + shift_ref[...], 0.0)


def kernel(x, y, ln_x_w, ln_x_b, ln_y_w, ln_y_b, w_qkv_x, w_qkv_y, w_dw_x,
           w_dw_y, t1, t2, t3, w_proj, norm_w, norm_b, w_ffn_in, w_ffn_dw,
           w_ffn_out, w_fuse1, b_fuse1, w_fuse2, b_fuse2, bn_w, bn_b):
    B, C, H, W = x.shape
    HW = H * W
    Ho, Wo = (H + 1) // 2, (W + 1) // 2
    HWs = Ho * Wo
    C3 = 3 * C
    hid = w_ffn_out.shape[0]
    hid2 = 2 * hid
    num_heads = t1.shape[0]
    hc = C // num_heads

    x2 = x.reshape(B, C, HW)
    y2 = y.reshape(B, C, HW)

    sel = np.zeros((HW, HWs), np.float32)
    pos = (2 * (np.arange(HWs) // Wo)) * W + 2 * (np.arange(HWs) % Wo)
    sel[pos, np.arange(HWs)] = 1.0
    selT = jnp.asarray(sel, jnp.bfloat16)

    hm = (np.arange(C)[:, None] // hc == np.arange(C)[None, :] // hc)
    hmask = jnp.asarray(hm.astype(np.float32))
    t1r = jnp.repeat(t1.reshape(num_heads), hc).reshape(C, 1)
    t2r = jnp.repeat(t2.reshape(num_heads), hc).reshape(C, 1)
    t3r = jnp.repeat(t3.reshape(num_heads), hc).reshape(C, 1)

    bf16 = jnp.bfloat16

    def dw_compose(w1, wdw):
        return jnp.concatenate(
            [w1 * wdw[k][None, :] for k in range(9)], axis=0).astype(bf16)

    wqx_c = dw_compose(w_qkv_x, w_dw_x)
    wqy_c = dw_compose(w_qkv_y, w_dw_y)
    wfi_c = dw_compose(w_ffn_in, w_ffn_dw)

    wfz_c = jnp.concatenate(
        [w_fuse1 @ w_fuse2[k] for k in range(9)], axis=0).astype(bf16)
    r_idx = np.arange(HW) // W
    c_idx = np.arange(HW) % W
    bias_rows = []
    for k in range(9):
        ki, kj = k // 3, k % 3
        valid = ((r_idx + ki - 1 >= 0) & (r_idx + ki - 1 < H)
                 & (c_idx + kj - 1 >= 0) & (c_idx + kj - 1 < W))
        bias_rows.append(valid.astype(np.float32))
    validity = jnp.asarray(np.stack(bias_rows, axis=1))
    tap_bias = jnp.stack([(b_fuse1 @ w_fuse2[k]).reshape(-1)
                          for k in range(9)], axis=0)
    bfz_field = validity @ tap_bias + b_fuse2

    wproj_b = w_proj.astype(bf16)
    wfo_b = w_ffn_out.astype(bf16)

    wspec = lambda *shape: pl.BlockSpec(shape, lambda b, s=shape: (0,) * len(s))
    nbspec = lambda *shape: pl.BlockSpec((NB,) + shape,
                                         lambda b, s=shape: (b,) + (0,) * len(s))

    kfn = functools.partial(_main_kernel, H=H, W=W)
    pad = HW + 2 * W

    z, stats = pl.pallas_call(
        kfn,
        out_shape=[jax.ShapeDtypeStruct((B, C, HW), jnp.float32),
                   jax.ShapeDtypeStruct((B, 8, C), jnp.float32)],
        grid=(B // NB,),
        in_specs=[
            nbspec(C, HW), nbspec(C, HW), wspec(HW, HWs),
            wspec(1, C), wspec(1, C), wspec(1, C), wspec(1, C),
            wspec(9 * C, C3), wspec(9 * C, C3),
            wspec(C, 1), wspec(C, 1), wspec(C, 1), wspec(C, C),
            wspec(C, C), wspec(1, C), wspec(1, C),
            wspec(9 * C, hid2), wspec(hid, C),
            wspec(9 * C, C), wspec(HW, C),
        ],
        out_specs=[nbspec(C, HW), nbspec(8, C)],
        scratch_shapes=[
            pltpu.VMEM((pad, C), jnp.bfloat16),
            pltpu.VMEM((pad, C), jnp.bfloat16),
            pltpu.VMEM((pad, C), jnp.bfloat16),
            pltpu.VMEM((NB * HW, 9 * C), jnp.bfloat16),
            pltpu.VMEM((pad, C), jnp.bfloat16),
            pltpu.VMEM((pad, C), jnp.bfloat16),
            pltpu.VMEM((pad, C), jnp.bfloat16),
            pltpu.VMEM((NB * HW, 9 * C), jnp.bfloat16),
        ],
        compiler_params=pltpu.CompilerParams(
            dimension_semantics=("parallel",),
            vmem_limit_bytes=48 * 1024 * 1024),
    )(x2, y2, selT,
      ln_x_w, ln_x_b, ln_y_w, ln_y_b,
      wqx_c, wqy_c,
      t1r, t2r, t3r, hmask,
      wproj_b, norm_w, norm_b,
      wfi_c, wfo_b,
      wfz_c, bfz_field)

    n = B * HW
    mean = jnp.sum(stats[:, 0, :], axis=0) / n
    var = jnp.maximum(jnp.sum(stats[:, 1, :], axis=0) / n - mean * mean, 0.0)
    inv = lax.rsqrt(var + 1e-5)
    bw = bn_w.reshape(-1)
    bb = bn_b.reshape(-1)
    scale = (bw * inv).reshape(C, 1)
    shift = (bb - mean * bw * inv).reshape(C, 1)

    out = pl.pallas_call(
        _bn_relu_kernel,
        out_shape=jax.ShapeDtypeStruct((B, C, HW), jnp.float32),
        grid=(B,),
        in_specs=[pl.BlockSpec((None, C, HW), lambda b: (b, 0, 0)),
                  pl.BlockSpec((C, 1), lambda b: (0, 0)),
                  pl.BlockSpec((C, 1), lambda b: (0, 0))],
        out_specs=pl.BlockSpec((None, C, HW), lambda b: (b, 0, 0)),
        compiler_params=pltpu.CompilerParams(dimension_semantics=("parallel",)),
    )(z, scale, shift)

    return out.reshape(B, C, H, W)


# speedup vs baseline: 1.1840x; 1.1199x over previous
import functools
import numpy as np
import jax
import jax.numpy as jnp
from jax import lax
from jax.experimental import pallas as pl
from jax.experimental.pallas import tpu as pltpu

NB = 4


def _ln_last(t, w, b):
    mu = jnp.mean(t, axis=-1, keepdims=True)
    var = jnp.mean(jnp.square(t - mu), axis=-1, keepdims=True)
    return (t - mu) * lax.rsqrt(var + 1e-5) * w + b


def _l2n_cols(v):
    ss = jnp.sum(v * v, axis=0, keepdims=True)
    return v * lax.rsqrt(jnp.maximum(ss, 1e-24))


def _softmax_rows(s):
    m = jnp.max(s, axis=-1, keepdims=True)
    e = jnp.exp(s - m)
    return e * pl.reciprocal(jnp.sum(e, axis=-1, keepdims=True), approx=True)


def _gelu(x):
    x2 = x * x
    u2 = x * (1.5957691216057308 + 0.07135481627613564 * x2)
    r = pl.reciprocal(jnp.exp(u2) + 1.0, approx=True)
    return x * (1.0 - r)


def _bdot(a, b):
    return jnp.dot(a.astype(jnp.bfloat16), b.astype(jnp.bfloat16),
                   preferred_element_type=jnp.float32)


def _bdot_t(a, b):
    return lax.dot_general(a.astype(jnp.bfloat16), b.astype(jnp.bfloat16),
                           dimension_numbers=(((1,), (1,)), ((), ())),
                           preferred_element_type=jnp.float32)


def _bdot_0(a, b):
    return lax.dot_general(a.astype(jnp.bfloat16), b.astype(jnp.bfloat16),
                           dimension_numbers=(((0,), (0,)), ((), ())),
                           preferred_element_type=jnp.float32)


def _patch9_multi(t2, cb, lb, rb, patch, mL, mR, HW, W, C):
    for i in range(NB):
        t = t2[i * HW:(i + 1) * HW, :]
        zhalo = jnp.zeros((W, C), jnp.bfloat16)
        cb[0:W, :] = zhalo
        cb[W + HW:2 * W + HW, :] = zhalo
        lb[0:W, :] = zhalo
        lb[W + HW:2 * W + HW, :] = zhalo
        rb[0:W, :] = zhalo
        rb[W + HW:2 * W + HW, :] = zhalo
        cb[W:W + HW, :] = t.astype(jnp.bfloat16)
        lb[W:W + HW, :] = cb[W - 1:W - 1 + HW, :] * mL
        rb[W:W + HW, :] = cb[W + 1:W + 1 + HW, :] * mR
        r0 = i * HW
        for ki in range(3):
            o = W * ki
            patch[r0:r0 + HW, (3 * ki + 0) * C:(3 * ki + 1) * C] = lb[o:o + HW, :]
            patch[r0:r0 + HW, (3 * ki + 1) * C:(3 * ki + 2) * C] = cb[o:o + HW, :]
            patch[r0:r0 + HW, (3 * ki + 2) * C:(3 * ki + 3) * C] = rb[o:o + HW, :]


def _main_kernel(
        x_ref, y_ref, selT_ref,
        lnxw_ref, lnxb_ref, lnyw_ref, lnyb_ref,
        wqx_ref, wqy_ref,
        t1r_ref, t2r_ref, t3r_ref, hmask_ref,
        wproj_ref, nw_ref, nb_ref,
        wfi_ref, wfo_ref,
        wfz_ref, bfz_ref,
        z_ref, stats_ref,
        cba, lba, rba, patcha, cbb, lbb, rbb, patchb,
        *, H, W):
    C = x_ref.shape[1]
    HW = x_ref.shape[2]
    M = NB * HW
    hid2 = wfi_ref.shape[1]
    hid = hid2 // 2

    col = lax.broadcasted_iota(jnp.int32, (HW, 1), 0) % W
    mL = (col != 0).astype(jnp.bfloat16)
    mR = (col != W - 1).astype(jnp.bfloat16)

    x_cl = jnp.transpose(x_ref[...], (0, 2, 1)).reshape(M, C)
    y_cl = jnp.transpose(y_ref[...], (0, 2, 1)).reshape(M, C)

    def branch(t_cl, lnw, lnb, wq_ref, cb, lb, rb, patch):
        tn = _ln_last(t_cl, lnw, lnb)
        _patch9_multi(tn, cb, lb, rb, patch, mL, mR, HW, W, C)
        return jnp.dot(patch[...], wq_ref[...],
                       preferred_element_type=jnp.float32)

    dwx = branch(x_cl, lnxw_ref[...], lnxb_ref[...], wqx_ref,
                 cba, lba, rba, patcha)
    dwy = branch(y_cl, lnyw_ref[...], lnyb_ref[...], wqy_ref,
                 cbb, lbb, rbb, patchb)

    selT = selT_ref[...]
    hmask = hmask_ref[...]
    t1r = t1r_ref[...]
    t2r = t2r_ref[...]
    t3r = t3r_ref[...]

    attn_rows = []
    for i in range(NB):
        dx = dwx[i * HW:(i + 1) * HW, :]
        dy = dwy[i * HW:(i + 1) * HW, :]
        qx, kx, vx = dx[:, 0:C], dx[:, C:2 * C], dx[:, 2 * C:3 * C]
        qy, ky, vy = dy[:, 0:C], dy[:, C:2 * C], dy[:, 2 * C:3 * C]

        kxs = _bdot_0(selT, kx)
        vxs = _bdot_0(selT, vx)
        qys = _bdot_0(selT, qy)

        qxn = _l2n_cols(qx)
        kyn = _l2n_cols(ky)
        qyn = _l2n_cols(qys)
        kxn = _l2n_cols(kxs)

        s1 = _bdot_0(qxn, kyn) * t1r
        attnx = _softmax_rows(jnp.where(hmask > 0.5, s1, -1e30))
        s2 = _bdot_0(qyn, kxn) * t2r
        attny = _softmax_rows(jnp.where(hmask > 0.5, s2, -1e30))

        av = _bdot_t(vxs, attny)
        t2v = _bdot_t(av, attnx)
        s3 = _bdot_0(t2v, vxs) * hmask * t3r
        attn_rows.append(_bdot_t(vy, s3))

    attn_cl = jnp.concatenate(attn_rows, axis=0)

    out = x_cl + _bdot(attn_cl, wproj_ref[...])

    on = _ln_last(out, nw_ref[...], nb_ref[...])
    _patch9_multi(on, cba, lba, rba, patcha, mL, mR, HW, W, C)
    dwf = jnp.dot(patcha[...], wfi_ref[...],
                  preferred_element_type=jnp.float32)
    g = _gelu(dwf[:, 0:hid]) * dwf[:, hid:hid2]
    out = out + _bdot(g, wfo_ref[...])

    z0 = x_cl + x_cl * out
    _patch9_multi(z0, cbb, lbb, rbb, patchb, mL, mR, HW, W, C)
    z2 = jnp.dot(patchb[...], wfz_ref[...],
                 preferred_element_type=jnp.float32)

    bfz = bfz_ref[...]
    for i in range(NB):
        zi = z2[i * HW:(i + 1) * HW, :] + bfz
        z_ref[i] = jnp.transpose(zi)
        s_sum = jnp.sum(zi, axis=0, keepdims=True)
        s_sq = jnp.sum(zi * zi, axis=0, keepdims=True)
        stats_ref[i] = jnp.concatenate(
            [s_sum, s_sq, jnp.zeros((6, C), jnp.float32)], axis=0)


def _bn_relu_kernel(z_ref, scale_ref, shift_ref, o_ref):
    o_ref[...] = jnp.maximum(
        z_ref[...] * scale_ref[...][None] + shift_ref[...][None], 0.0)


def kernel(x, y, ln_x_w, ln_x_b, ln_y_w, ln_y_b, w_qkv_x, w_qkv_y, w_dw_x,
           w_dw_y, t1, t2, t3, w_proj, norm_w, norm_b, w_ffn_in, w_ffn_dw,
           w_ffn_out, w_fuse1, b_fuse1, w_fuse2, b_fuse2, bn_w, bn_b):
    B, C, H, W = x.shape
    HW = H * W
    Ho, Wo = (H + 1) // 2, (W + 1) // 2
    HWs = Ho * Wo
    C3 = 3 * C
    hid = w_ffn_out.shape[0]
    hid2 = 2 * hid
    num_heads = t1.shape[0]
    hc = C // num_heads

    x2 = x.reshape(B, C, HW)
    y2 = y.reshape(B, C, HW)

    sel = np.zeros((HW, HWs), np.float32)
    pos = (2 * (np.arange(HWs) // Wo)) * W + 2 * (np.arange(HWs) % Wo)
    sel[pos, np.arange(HWs)] = 1.0
    selT = jnp.asarray(sel, jnp.bfloat16)

    hm = (np.arange(C)[:, None] // hc == np.arange(C)[None, :] // hc)
    hmask = jnp.asarray(hm.astype(np.float32))
    t1r = jnp.repeat(t1.reshape(num_heads), hc).reshape(C, 1)
    t2r = jnp.repeat(t2.reshape(num_heads), hc).reshape(C, 1)
    t3r = jnp.repeat(t3.reshape(num_heads), hc).reshape(C, 1)

    bf16 = jnp.bfloat16

    def dw_compose(w1, wdw):
        return jnp.concatenate(
            [w1 * wdw[k][None, :] for k in range(9)], axis=0).astype(bf16)

    wqx_c = dw_compose(w_qkv_x, w_dw_x)
    wqy_c = dw_compose(w_qkv_y, w_dw_y)
    wfi_c = dw_compose(w_ffn_in, w_ffn_dw)

    wfz_c = jnp.concatenate(
        [w_fuse1 @ w_fuse2[k] for k in range(9)], axis=0).astype(bf16)
    r_idx = np.arange(HW) // W
    c_idx = np.arange(HW) % W
    bias_rows = []
    for k in range(9):
        ki, kj = k // 3, k % 3
        valid = ((r_idx + ki - 1 >= 0) & (r_idx + ki - 1 < H)
                 & (c_idx + kj - 1 >= 0) & (c_idx + kj - 1 < W))
        bias_rows.append(valid.astype(np.float32))
    validity = jnp.asarray(np.stack(bias_rows, axis=1))
    tap_bias = jnp.stack([(b_fuse1 @ w_fuse2[k]).reshape(-1)
                          for k in range(9)], axis=0)
    bfz_field = validity @ tap_bias + b_fuse2

    wproj_b = w_proj.astype(bf16)
    wfo_b = w_ffn_out.astype(bf16)

    wspec = lambda *shape: pl.BlockSpec(shape, lambda b, s=shape: (0,) * len(s))
    nbspec = lambda *shape: pl.BlockSpec((NB,) + shape,
                                         lambda b, s=shape: (b,) + (0,) * len(s))

    kfn = functools.partial(_main_kernel, H=H, W=W)
    pad = HW + 2 * W

    z, stats = pl.pallas_call(
        kfn,
        out_shape=[jax.ShapeDtypeStruct((B, C, HW), jnp.float32),
                   jax.ShapeDtypeStruct((B, 8, C), jnp.float32)],
        grid=(B // NB,),
        in_specs=[
            nbspec(C, HW), nbspec(C, HW), wspec(HW, HWs),
            wspec(1, C), wspec(1, C), wspec(1, C), wspec(1, C),
            wspec(9 * C, C3), wspec(9 * C, C3),
            wspec(C, 1), wspec(C, 1), wspec(C, 1), wspec(C, C),
            wspec(C, C), wspec(1, C), wspec(1, C),
            wspec(9 * C, hid2), wspec(hid, C),
            wspec(9 * C, C), wspec(HW, C),
        ],
        out_specs=[nbspec(C, HW), nbspec(8, C)],
        scratch_shapes=[
            pltpu.VMEM((pad, C), jnp.bfloat16),
            pltpu.VMEM((pad, C), jnp.bfloat16),
            pltpu.VMEM((pad, C), jnp.bfloat16),
            pltpu.VMEM((NB * HW, 9 * C), jnp.bfloat16),
            pltpu.VMEM((pad, C), jnp.bfloat16),
            pltpu.VMEM((pad, C), jnp.bfloat16),
            pltpu.VMEM((pad, C), jnp.bfloat16),
            pltpu.VMEM((NB * HW, 9 * C), jnp.bfloat16),
        ],
        compiler_params=pltpu.CompilerParams(
            dimension_semantics=("parallel",),
            vmem_limit_bytes=48 * 1024 * 1024),
    )(x2, y2, selT,
      ln_x_w, ln_x_b, ln_y_w, ln_y_b,
      wqx_c, wqy_c,
      t1r, t2r, t3r, hmask,
      wproj_b, norm_w, norm_b,
      wfi_c, wfo_b,
      wfz_c, bfz_field)

    n = B * HW
    mean = jnp.sum(stats[:, 0, :], axis=0) / n
    var = jnp.maximum(jnp.sum(stats[:, 1, :], axis=0) / n - mean * mean, 0.0)
    inv = lax.rsqrt(var + 1e-5)
    bw = bn_w.reshape(-1)
    bb = bn_b.reshape(-1)
    scale = (bw * inv).reshape(C, 1)
    shift = (bb - mean * bw * inv).reshape(C, 1)

    NBB = 16
    out = pl.pallas_call(
        _bn_relu_kernel,
        out_shape=jax.ShapeDtypeStruct((B, C, HW), jnp.float32),
        grid=(B // NBB,),
        in_specs=[pl.BlockSpec((NBB, C, HW), lambda b: (b, 0, 0)),
                  pl.BlockSpec((C, 1), lambda b: (0, 0)),
                  pl.BlockSpec((C, 1), lambda b: (0, 0))],
        out_specs=pl.BlockSpec((NBB, C, HW), lambda b: (b, 0, 0)),
        compiler_params=pltpu.CompilerParams(dimension_semantics=("parallel",)),
    )(z, scale, shift)

    return out.reshape(B, C, H, W)
